# baseline (speedup 1.0000x reference)
import jax
import jax.numpy as jnp
from jax import lax
from jax.experimental import pallas as pl
from jax.experimental.pallas import tpu as pltpu
from jax.experimental.pallas import tpu_sc as plsc

NC = 2
NS = 16
CHUNK = 64
HALF = 128

_MESH = dict(core_axis_name="c", subcore_axis_name="s")


def _sc_degree_hist(dst3h, z128, ones128, n, n_pad, e_pad):
    nch = e_pad // (NC * NS) // CHUNK
    zrows = n_pad // NS

    def body(dst_hbm, z_hbm, ones_hbm, out_hbm, hist_sh, dstv, ones_v, sem,
             sems):
        c = lax.axis_index("c")
        s = lax.axis_index("s")
        pltpu.sync_copy(dst_hbm.at[c * NS + s], dstv)
        pltpu.sync_copy(ones_hbm, ones_v)
        pltpu.sync_copy(z_hbm.at[pl.ds(s * zrows, zrows), :],
                        hist_sh.at[pl.ds(s * zrows, zrows), :])
        plsc.subcore_barrier()

        for k in range(nch):
            pltpu.async_copy(ones_v, hist_sh.at[dstv.at[k]], sems, add=True)
        for k in range(nch):
            pltpu.make_async_copy(ones_v, hist_sh.at[dstv.at[k]], sems).wait()

        plsc.subcore_barrier()
        pltpu.sync_copy(hist_sh.at[pl.ds(s * zrows, zrows), :],
                        out_hbm.at[c, pl.ds(s * zrows, zrows), :])

    kern = pl.kernel(
        body,
        out_type=jax.ShapeDtypeStruct((NC, n_pad, HALF), jnp.float32),
        mesh=plsc.VectorSubcoreMesh(**_MESH),
        scratch_types=[
            pltpu.VMEM_SHARED((n_pad, HALF), jnp.float32),
            pltpu.VMEM((nch, CHUNK), jnp.int32),
            pltpu.VMEM((CHUNK, HALF), jnp.float32),
            pltpu.SemaphoreType.DMA,
            pltpu.SemaphoreType.DMA,
        ],
    )
    return kern(dst3h, z128, ones128)


def _sc_edge_scatter(glo, ghi, ed4, z128, n, n_pad, e_pad):
    nch = e_pad // NS // CHUNK
    zrows = n_pad // NS
    assert nch % 8 == 0

    def body(glo_hbm, ghi_hbm, ed_hbm, z_hbm, out_hbm, acc_sh, idxv, rows,
             si0, si1, si2, si3, si4, si5, si6, si7,
             sg0, sg1, sg2, sg3, ss0, ss1, ss2, ss3):
        sis = [si0, si1, si2, si3, si4, si5, si6, si7]
        sgs = [sg0, sg1, sg2, sg3]
        sss = [ss0, ss1, ss2, ss3]
        c = lax.axis_index("c")
        s = lax.axis_index("s")
        pltpu.sync_copy(z_hbm.at[pl.ds(s * zrows, zrows), :],
                        acc_sh.at[pl.ds(s * zrows, zrows), :])
        plsc.subcore_barrier()

        def run_half(g_hbm):
            def ii(k, ib):
                pltpu.async_copy(ed_hbm.at[s, k], idxv.at[ib], sis[ib])

            def iw(k, ib):
                pltpu.make_async_copy(ed_hbm.at[s, k], idxv.at[ib],
                                      sis[ib]).wait()

            def gi(k, rb, ib):
                pltpu.async_copy(g_hbm.at[idxv.at[ib, 0]], rows.at[rb],
                                 sgs[rb])

            def gw(k, rb, ib):
                pltpu.make_async_copy(g_hbm.at[idxv.at[ib, 0]], rows.at[rb],
                                      sgs[rb]).wait()

            def sci(k, rb, ib):
                pltpu.async_copy(rows.at[rb], acc_sh.at[idxv.at[ib, 1]],
                                 sss[rb], add=True)

            def scw(k, rb, ib):
                pltpu.make_async_copy(rows.at[rb], acc_sh.at[idxv.at[ib, 1]],
                                      sss[rb]).wait()

            for t in range(5):
                ii(t, t)
            iw(0, 0)
            gi(0, 0, 0)

            @pl.loop(0, nch, step=8)
            def _(g0):
                for j in range(8):
                    k = g0 + j
                    gw(k, j % 4, j)
                    sci(k, j % 4, j)

                    @pl.when(k >= 3)
                    def _():
                        scw(k - 3, (j + 1) % 4, (j + 5) % 8)

                    @pl.when(k + 5 < nch)
                    def _():
                        ii(k + 5, (j + 5) % 8)

                    @pl.when(k + 1 < nch)
                    def _():
                        iw(k + 1, (j + 1) % 8)
                        gi(k + 1, (j + 1) % 4, (j + 1) % 8)

            for t in range(3):
                k = nch - 3 + t
                scw(k, k % 4, k % 8)

        @pl.when(c == 0)
        def _():
            run_half(glo_hbm)

        @pl.when(c == 1)
        def _():
            run_half(ghi_hbm)

        plsc.subcore_barrier()
        pltpu.sync_copy(acc_sh.at[pl.ds(s * zrows, zrows), :],
                        out_hbm.at[c, pl.ds(s * zrows, zrows), :])

    kern = pl.kernel(
        body,
        out_type=jax.ShapeDtypeStruct((NC, n_pad, HALF), jnp.float32),
        mesh=plsc.VectorSubcoreMesh(**_MESH),
        scratch_types=[
            pltpu.VMEM_SHARED((n_pad, HALF), jnp.float32),
            pltpu.VMEM((8, 2, CHUNK), jnp.int32),
            pltpu.VMEM((4, CHUNK, HALF), jnp.float32),
        ] + [pltpu.SemaphoreType.DMA] * 16,
    )
    return kern(glo, ghi, ed4, z128)


def _tc_matmul(x, w, bn):
    n, din = x.shape
    dout = w.shape[1]

    def body(x_ref, w_ref, h_ref):
        h_ref[...] = jnp.dot(x_ref[...], w_ref[...],
                             preferred_element_type=jnp.float32)

    return pl.pallas_call(
        body,
        grid=(n // bn,),
        in_specs=[pl.BlockSpec((bn, din), lambda i: (i, 0)),
                  pl.BlockSpec((din, dout), lambda i: (0, 0))],
        out_specs=pl.BlockSpec((bn, dout), lambda i: (i, 0)),
        out_shape=jax.ShapeDtypeStruct((n, dout), jnp.float32),
    )(x, w)


def _tc_scale(h, hist, bn):
    n, d = h.shape

    def body(h_ref, hist_ref, dis_ref, glo_ref, ghi_ref):
        deg = 1.0 + hist_ref[0, :, 0:1] + hist_ref[1, :, 0:1]
        dis = lax.rsqrt(deg)
        dis_ref[...] = dis
        g = h_ref[...] * dis
        glo_ref[...] = g[:, :HALF]
        ghi_ref[...] = g[:, HALF:]

    return pl.pallas_call(
        body,
        grid=(n // bn,),
        in_specs=[pl.BlockSpec((bn, d), lambda i: (i, 0)),
                  pl.BlockSpec((NC, bn, HALF), lambda i: (0, i, 0))],
        out_specs=[pl.BlockSpec((bn, 1), lambda i: (i, 0)),
                   pl.BlockSpec((bn, HALF), lambda i: (i, 0)),
                   pl.BlockSpec((bn, HALF), lambda i: (i, 0))],
        out_shape=[jax.ShapeDtypeStruct((n, 1), jnp.float32),
                   jax.ShapeDtypeStruct((n, HALF), jnp.float32),
                   jax.ShapeDtypeStruct((n, HALF), jnp.float32)],
    )(h, hist)


def _tc_layer2(s1, glo, ghi, dis, b1, w2, batch3, g_pool, bn):
    n = dis.shape[0]
    d = w2.shape[0]
    nblk = n // bn

    def body(s_ref, glo_ref, ghi_ref, dis_ref, b_ref, w_ref, batch_ref,
             p1_ref, g2lo_ref, g2hi_ref, acc):
        i = pl.program_id(0)

        @pl.when(i == 0)
        def _():
            acc[...] = jnp.zeros_like(acc)

        dis = dis_ref[...]
        lo = dis * (s_ref[0] + glo_ref[...]) + b_ref[:, :HALF]
        hi = dis * (s_ref[1] + ghi_ref[...]) + b_ref[:, HALF:]
        x1 = jax.nn.relu(jnp.concatenate([lo, hi], axis=1))
        mask = (lax.broadcasted_iota(jnp.int32, (g_pool, bn), 0)
                == batch_ref[0]).astype(jnp.float32)
        acc[...] += jnp.dot(mask, x1, preferred_element_type=jnp.float32)
        h2 = jnp.dot(x1, w_ref[...], preferred_element_type=jnp.float32)
        g2 = dis * h2
        g2lo_ref[...] = g2[:, :HALF]
        g2hi_ref[...] = g2[:, HALF:]

        @pl.when(i == nblk - 1)
        def _():
            p1_ref[...] = acc[...]

    return pl.pallas_call(
        body,
        grid=(nblk,),
        in_specs=[pl.BlockSpec((NC, bn, HALF), lambda i: (0, i, 0)),
                  pl.BlockSpec((bn, HALF), lambda i: (i, 0)),
                  pl.BlockSpec((bn, HALF), lambda i: (i, 0)),
                  pl.BlockSpec((bn, 1), lambda i: (i, 0)),
                  pl.BlockSpec((1, d), lambda i: (0, 0)),
                  pl.BlockSpec((d, d), lambda i: (0, 0)),
                  pl.BlockSpec((1, 1, bn), lambda i: (i, 0, 0))],
        out_specs=[pl.BlockSpec((g_pool, d), lambda i: (0, 0)),
                   pl.BlockSpec((bn, HALF), lambda i: (i, 0)),
                   pl.BlockSpec((bn, HALF), lambda i: (i, 0))],
        out_shape=[jax.ShapeDtypeStruct((g_pool, d), jnp.float32),
                   jax.ShapeDtypeStruct((n, HALF), jnp.float32),
                   jax.ShapeDtypeStruct((n, HALF), jnp.float32)],
        scratch_shapes=[pltpu.VMEM((g_pool, d), jnp.float32)],
    )(s1, glo, ghi, dis, b1, w2, batch3)


def _tc_final(s2, glo, ghi, dis, b2, batch3, p1, wl1, bl1, wl2, bl2,
              g_pool, bn):
    n = dis.shape[0]
    d = 2 * HALF
    nblk = n // bn
    hmid = wl1.shape[1]
    p_out = wl2.shape[1]

    def body(s_ref, glo_ref, ghi_ref, dis_ref, b_ref, batch_ref, p1_ref,
             wl1_ref, bl1_ref, wl2_ref, bl2_ref, out_ref, acc):
        i = pl.program_id(0)

        @pl.when(i == 0)
        def _():
            acc[...] = jnp.zeros_like(acc)

        dis = dis_ref[...]
        lo = dis * (s_ref[0] + glo_ref[...]) + b_ref[:, :HALF]
        hi = dis * (s_ref[1] + ghi_ref[...]) + b_ref[:, HALF:]
        x2 = jax.nn.relu(jnp.concatenate([lo, hi], axis=1))
        mask = (lax.broadcasted_iota(jnp.int32, (g_pool, bn), 0)
                == batch_ref[0]).astype(jnp.float32)
        acc[...] += jnp.dot(mask, x2, preferred_element_type=jnp.float32)

        @pl.when(i == nblk - 1)
        def _():
            pooled = jnp.concatenate([p1_ref[...], acc[...]], axis=1)
            h = jnp.dot(pooled, wl1_ref[...],
                        preferred_element_type=jnp.float32) + bl1_ref[...]
            out_ref[...] = jnp.dot(h, wl2_ref[...],
                                   preferred_element_type=jnp.float32) + bl2_ref[...]

    return pl.pallas_call(
        body,
        grid=(nblk,),
        in_specs=[pl.BlockSpec((NC, bn, HALF), lambda i: (0, i, 0)),
                  pl.BlockSpec((bn, HALF), lambda i: (i, 0)),
                  pl.BlockSpec((bn, HALF), lambda i: (i, 0)),
                  pl.BlockSpec((bn, 1), lambda i: (i, 0)),
                  pl.BlockSpec((1, d), lambda i: (0, 0)),
                  pl.BlockSpec((1, 1, bn), lambda i: (i, 0, 0)),
                  pl.BlockSpec((g_pool, d), lambda i: (0, 0)),
                  pl.BlockSpec((2 * d, hmid), lambda i: (0, 0)),
                  pl.BlockSpec((1, hmid), lambda i: (0, 0)),
                  pl.BlockSpec((hmid, p_out), lambda i: (0, 0)),
                  pl.BlockSpec((1, p_out), lambda i: (0, 0))],
        out_specs=pl.BlockSpec((g_pool, p_out), lambda i: (0, 0)),
        out_shape=jax.ShapeDtypeStruct((g_pool, p_out), jnp.float32),
        scratch_shapes=[pltpu.VMEM((g_pool, d), jnp.float32)],
    )(s2, glo, ghi, dis, b2, batch3, p1, wl1, bl1, wl2, bl2)


def kernel(x, edge_index, batch, W1g, b1g, W2g, b2g, Wl1, bl1, Wl2, bl2):
    n, din = x.shape
    e = edge_index.shape[1]
    g_pool = Wl2.shape[1]
    bn = 1000

    n_pad = ((n + NS * 8 - 1) // (NS * 8)) * (NS * 8)
    unit = NS * CHUNK * 8
    e_pad = ((e + unit - 1) // unit) * unit

    src = edge_index[0].astype(jnp.int32)
    dst = edge_index[1].astype(jnp.int32)
    srcp = jnp.concatenate([src, jnp.zeros((e_pad - e,), jnp.int32)])
    dstp = jnp.concatenate([dst, jnp.full((e_pad - e,), n, jnp.int32)])
    nch = e_pad // NS // CHUNK
    ed4 = jnp.stack([srcp.reshape(NS, nch, CHUNK),
                     dstp.reshape(NS, nch, CHUNK)], axis=2)
    dst3h = dstp.reshape(NC * NS, nch // NC, CHUNK)

    z128 = jnp.zeros((n_pad, HALF), jnp.float32)
    ones128 = jnp.ones((CHUNK, HALF), jnp.float32)
    batch3 = batch.astype(jnp.int32).reshape(n // bn, 1, bn)
    b1r = b1g.reshape(1, -1)
    b2r = b2g.reshape(1, -1)
    bl1r = bl1.reshape(1, -1)
    bl2r = bl2.reshape(1, -1)

    hist = _sc_degree_hist(dst3h, z128, ones128, n, n_pad, e_pad)
    h1 = _tc_matmul(x, W1g, bn)
    dis, g1lo, g1hi = _tc_scale(h1, hist, bn)
    s1 = _sc_edge_scatter(g1lo, g1hi, ed4, z128, n, n_pad, e_pad)
    p1, g2lo, g2hi = _tc_layer2(s1, g1lo, g1hi, dis, b1r, W2g, batch3,
                                g_pool, bn)
    s2 = _sc_edge_scatter(g2lo, g2hi, ed4, z128, n, n_pad, e_pad)
    out = _tc_final(s2, g2lo, g2hi, dis, b2r, batch3, p1,
                    Wl1, bl1r, Wl2, bl2r, g_pool, bn)
    return out


# speedup vs baseline: 1.2035x; 1.2035x over previous
import jax
import jax.numpy as jnp
from jax import lax
from jax.experimental import pallas as pl
from jax.experimental.pallas import tpu as pltpu
from jax.experimental.pallas import tpu_sc as plsc

NC = 2
NS = 16
CHUNK = 128
HALF = 128

_MESH = dict(core_axis_name="c", subcore_axis_name="s")


def _sc_degree_hist(dst3h, z128, ones128, n, n_pad, e_pad):
    nch = e_pad // (NC * NS) // CHUNK
    zrows = n_pad // NS

    def body(dst_hbm, z_hbm, ones_hbm, out_hbm, hist_sh, dstv, ones_v, sem,
             sems):
        c = lax.axis_index("c")
        s = lax.axis_index("s")
        pltpu.sync_copy(dst_hbm.at[c * NS + s], dstv)
        pltpu.sync_copy(ones_hbm, ones_v)
        pltpu.sync_copy(z_hbm.at[pl.ds(s * zrows, zrows), :],
                        hist_sh.at[pl.ds(s * zrows, zrows), :])
        plsc.subcore_barrier()

        for k in range(nch):
            pltpu.async_copy(ones_v, hist_sh.at[dstv.at[k]], sems, add=True)
        for k in range(nch):
            pltpu.make_async_copy(ones_v, hist_sh.at[dstv.at[k]], sems).wait()

        plsc.subcore_barrier()
        pltpu.sync_copy(hist_sh.at[pl.ds(s * zrows, zrows), :],
                        out_hbm.at[c, pl.ds(s * zrows, zrows), :])

    kern = pl.kernel(
        body,
        out_type=jax.ShapeDtypeStruct((NC, n_pad, HALF), jnp.float32),
        mesh=plsc.VectorSubcoreMesh(**_MESH),
        scratch_types=[
            pltpu.VMEM_SHARED((n_pad, HALF), jnp.float32),
            pltpu.VMEM((nch, CHUNK), jnp.int32),
            pltpu.VMEM((CHUNK, HALF), jnp.float32),
            pltpu.SemaphoreType.DMA,
            pltpu.SemaphoreType.DMA,
        ],
    )
    return kern(dst3h, z128, ones128)


def _sc_edge_scatter(glo, ghi, ed4, z128, n, n_pad, e_pad):
    nch = e_pad // NS // CHUNK
    zrows = n_pad // NS
    assert nch % 4 == 0

    def body(glo_hbm, ghi_hbm, ed_hbm, z_hbm, out_hbm,
             acc_sh, idxv, rows, si0, si1, si2, si3, sg0, sg1, ss0, ss1):
        sis = [si0, si1, si2, si3]
        sgs = [sg0, sg1]
        sss = [ss0, ss1]
        c = lax.axis_index("c")
        s = lax.axis_index("s")
        pltpu.sync_copy(z_hbm.at[pl.ds(s * zrows, zrows), :],
                        acc_sh.at[pl.ds(s * zrows, zrows), :])
        plsc.subcore_barrier()

        def run_half(g_hbm):
            def ii(k, ib):
                pltpu.async_copy(ed_hbm.at[s, k], idxv.at[ib], sis[ib])

            def iw(k, ib):
                pltpu.make_async_copy(ed_hbm.at[s, k], idxv.at[ib],
                                      sis[ib]).wait()

            def gi(k, rb, ib):
                pltpu.async_copy(g_hbm.at[idxv.at[ib, 0]], rows.at[rb],
                                 sgs[rb])

            def gw(k, rb, ib):
                pltpu.make_async_copy(g_hbm.at[idxv.at[ib, 0]], rows.at[rb],
                                      sgs[rb]).wait()

            def sci(k, rb, ib):
                pltpu.async_copy(rows.at[rb], acc_sh.at[idxv.at[ib, 1]],
                                 sss[rb], add=True)

            def scw(k, rb, ib):
                pltpu.make_async_copy(rows.at[rb], acc_sh.at[idxv.at[ib, 1]],
                                      sss[rb]).wait()

            for t in range(3):
                ii(t, t)
            iw(0, 0)
            gi(0, 0, 0)

            @pl.loop(0, nch, step=4)
            def _(g0):
                for j in range(4):
                    k = g0 + j

                    @pl.when(k >= 1)
                    def _():
                        scw(k - 1, (j + 1) % 2, (j + 3) % 4)

                    @pl.when(k + 1 < nch)
                    def _():
                        iw(k + 1, (j + 1) % 4)
                        gi(k + 1, (j + 1) % 2, (j + 1) % 4)

                    gw(k, j % 2, j)
                    sci(k, j % 2, j)

                    @pl.when(k + 3 < nch)
                    def _():
                        ii(k + 3, (j + 3) % 4)

            scw(nch - 1, (nch - 1) % 2, (nch - 1) % 4)

        @pl.when(c == 0)
        def _():
            run_half(glo_hbm)

        @pl.when(c == 1)
        def _():
            run_half(ghi_hbm)

        plsc.subcore_barrier()
        pltpu.sync_copy(acc_sh.at[pl.ds(s * zrows, zrows), :],
                        out_hbm.at[c, pl.ds(s * zrows, zrows), :])

    kern = pl.kernel(
        body,
        out_type=jax.ShapeDtypeStruct((NC, n_pad, HALF), jnp.float32),
        mesh=plsc.VectorSubcoreMesh(**_MESH),
        scratch_types=[
            pltpu.VMEM_SHARED((n_pad, HALF), jnp.float32),
            pltpu.VMEM((4, 2, CHUNK), jnp.int32),
            pltpu.VMEM((2, CHUNK, HALF), jnp.float32),
        ] + [pltpu.SemaphoreType.DMA] * 8,
    )
    return kern(glo, ghi, ed4, z128)


def _tc_matmul(x, w, bn):
    n, din = x.shape
    dout = w.shape[1]

    def body(x_ref, w_ref, h_ref):
        h_ref[...] = jnp.dot(x_ref[...], w_ref[...],
                             preferred_element_type=jnp.float32)

    return pl.pallas_call(
        body,
        grid=(n // bn,),
        in_specs=[pl.BlockSpec((bn, din), lambda i: (i, 0)),
                  pl.BlockSpec((din, dout), lambda i: (0, 0))],
        out_specs=pl.BlockSpec((bn, dout), lambda i: (i, 0)),
        out_shape=jax.ShapeDtypeStruct((n, dout), jnp.float32),
    )(x, w)


def _tc_scale(h, hist, bn):
    n, d = h.shape

    def body(h_ref, hist_ref, dis_ref, glo_ref, ghi_ref):
        deg = 1.0 + hist_ref[0, :, 0:1] + hist_ref[1, :, 0:1]
        dis = lax.rsqrt(deg)
        dis_ref[...] = dis
        g = h_ref[...] * dis
        glo_ref[...] = g[:, :HALF]
        ghi_ref[...] = g[:, HALF:]

    return pl.pallas_call(
        body,
        grid=(n // bn,),
        in_specs=[pl.BlockSpec((bn, d), lambda i: (i, 0)),
                  pl.BlockSpec((NC, bn, HALF), lambda i: (0, i, 0))],
        out_specs=[pl.BlockSpec((bn, 1), lambda i: (i, 0)),
                   pl.BlockSpec((bn, HALF), lambda i: (i, 0)),
                   pl.BlockSpec((bn, HALF), lambda i: (i, 0))],
        out_shape=[jax.ShapeDtypeStruct((n, 1), jnp.float32),
                   jax.ShapeDtypeStruct((n, HALF), jnp.float32),
                   jax.ShapeDtypeStruct((n, HALF), jnp.float32)],
    )(h, hist)


def _tc_layer2(s1, glo, ghi, dis, b1, w2, batch3, g_pool, bn):
    n = dis.shape[0]
    d = w2.shape[0]
    nblk = n // bn

    def body(s_ref, glo_ref, ghi_ref, dis_ref, b_ref, w_ref, batch_ref,
             p1_ref, g2lo_ref, g2hi_ref, acc):
        i = pl.program_id(0)

        @pl.when(i == 0)
        def _():
            acc[...] = jnp.zeros_like(acc)

        dis = dis_ref[...]
        lo = dis * (s_ref[0] + glo_ref[...]) + b_ref[:, :HALF]
        hi = dis * (s_ref[1] + ghi_ref[...]) + b_ref[:, HALF:]
        x1 = jax.nn.relu(jnp.concatenate([lo, hi], axis=1))
        mask = (lax.broadcasted_iota(jnp.int32, (g_pool, bn), 0)
                == batch_ref[0]).astype(jnp.float32)
        acc[...] += jnp.dot(mask, x1, preferred_element_type=jnp.float32)
        h2 = jnp.dot(x1, w_ref[...], preferred_element_type=jnp.float32)
        g2 = dis * h2
        g2lo_ref[...] = g2[:, :HALF]
        g2hi_ref[...] = g2[:, HALF:]

        @pl.when(i == nblk - 1)
        def _():
            p1_ref[...] = acc[...]

    return pl.pallas_call(
        body,
        grid=(nblk,),
        in_specs=[pl.BlockSpec((NC, bn, HALF), lambda i: (0, i, 0)),
                  pl.BlockSpec((bn, HALF), lambda i: (i, 0)),
                  pl.BlockSpec((bn, HALF), lambda i: (i, 0)),
                  pl.BlockSpec((bn, 1), lambda i: (i, 0)),
                  pl.BlockSpec((1, d), lambda i: (0, 0)),
                  pl.BlockSpec((d, d), lambda i: (0, 0)),
                  pl.BlockSpec((1, 1, bn), lambda i: (i, 0, 0))],
        out_specs=[pl.BlockSpec((g_pool, d), lambda i: (0, 0)),
                   pl.BlockSpec((bn, HALF), lambda i: (i, 0)),
                   pl.BlockSpec((bn, HALF), lambda i: (i, 0))],
        out_shape=[jax.ShapeDtypeStruct((g_pool, d), jnp.float32),
                   jax.ShapeDtypeStruct((n, HALF), jnp.float32),
                   jax.ShapeDtypeStruct((n, HALF), jnp.float32)],
        scratch_shapes=[pltpu.VMEM((g_pool, d), jnp.float32)],
    )(s1, glo, ghi, dis, b1, w2, batch3)


def _tc_final(s2, glo, ghi, dis, b2, batch3, p1, wl1, bl1, wl2, bl2,
              g_pool, bn):
    n = dis.shape[0]
    d = 2 * HALF
    nblk = n // bn
    hmid = wl1.shape[1]
    p_out = wl2.shape[1]

    def body(s_ref, glo_ref, ghi_ref, dis_ref, b_ref, batch_ref, p1_ref,
             wl1_ref, bl1_ref, wl2_ref, bl2_ref, out_ref, acc):
        i = pl.program_id(0)

        @pl.when(i == 0)
        def _():
            acc[...] = jnp.zeros_like(acc)

        dis = dis_ref[...]
        lo = dis * (s_ref[0] + glo_ref[...]) + b_ref[:, :HALF]
        hi = dis * (s_ref[1] + ghi_ref[...]) + b_ref[:, HALF:]
        x2 = jax.nn.relu(jnp.concatenate([lo, hi], axis=1))
        mask = (lax.broadcasted_iota(jnp.int32, (g_pool, bn), 0)
                == batch_ref[0]).astype(jnp.float32)
        acc[...] += jnp.dot(mask, x2, preferred_element_type=jnp.float32)

        @pl.when(i == nblk - 1)
        def _():
            pooled = jnp.concatenate([p1_ref[...], acc[...]], axis=1)
            h = jnp.dot(pooled, wl1_ref[...],
                        preferred_element_type=jnp.float32) + bl1_ref[...]
            out_ref[...] = jnp.dot(h, wl2_ref[...],
                                   preferred_element_type=jnp.float32) + bl2_ref[...]

    return pl.pallas_call(
        body,
        grid=(nblk,),
        in_specs=[pl.BlockSpec((NC, bn, HALF), lambda i: (0, i, 0)),
                  pl.BlockSpec((bn, HALF), lambda i: (i, 0)),
                  pl.BlockSpec((bn, HALF), lambda i: (i, 0)),
                  pl.BlockSpec((bn, 1), lambda i: (i, 0)),
                  pl.BlockSpec((1, d), lambda i: (0, 0)),
                  pl.BlockSpec((1, 1, bn), lambda i: (i, 0, 0)),
                  pl.BlockSpec((g_pool, d), lambda i: (0, 0)),
                  pl.BlockSpec((2 * d, hmid), lambda i: (0, 0)),
                  pl.BlockSpec((1, hmid), lambda i: (0, 0)),
                  pl.BlockSpec((hmid, p_out), lambda i: (0, 0)),
                  pl.BlockSpec((1, p_out), lambda i: (0, 0))],
        out_specs=pl.BlockSpec((g_pool, p_out), lambda i: (0, 0)),
        out_shape=jax.ShapeDtypeStruct((g_pool, p_out), jnp.float32),
        scratch_shapes=[pltpu.VMEM((g_pool, d), jnp.float32)],
    )(s2, glo, ghi, dis, b2, batch3, p1, wl1, bl1, wl2, bl2)


def kernel(x, edge_index, batch, W1g, b1g, W2g, b2g, Wl1, bl1, Wl2, bl2):
    n, din = x.shape
    e = edge_index.shape[1]
    g_pool = Wl2.shape[1]
    bn = 1000

    n_pad = ((n + NS * 8 - 1) // (NS * 8)) * (NS * 8)
    unit = NS * CHUNK * 8
    e_pad = ((e + unit - 1) // unit) * unit

    src = edge_index[0].astype(jnp.int32)
    dst = edge_index[1].astype(jnp.int32)
    srcp = jnp.concatenate([src, jnp.zeros((e_pad - e,), jnp.int32)])
    dstp = jnp.concatenate([dst, jnp.full((e_pad - e,), n, jnp.int32)])
    nch = e_pad // NS // CHUNK
    ed4 = jnp.stack([srcp.reshape(NS, nch, CHUNK),
                     dstp.reshape(NS, nch, CHUNK)], axis=2)
    dst3h = dstp.reshape(NC * NS, nch // NC, CHUNK)

    z128 = jnp.zeros((n_pad, HALF), jnp.float32)
    ones128 = jnp.ones((CHUNK, HALF), jnp.float32)
    batch3 = batch.astype(jnp.int32).reshape(n // bn, 1, bn)
    b1r = b1g.reshape(1, -1)
    b2r = b2g.reshape(1, -1)
    bl1r = bl1.reshape(1, -1)
    bl2r = bl2.reshape(1, -1)

    hist = _sc_degree_hist(dst3h, z128, ones128, n, n_pad, e_pad)
    h1 = _tc_matmul(x, W1g, bn)
    dis, g1lo, g1hi = _tc_scale(h1, hist, bn)
    s1 = _sc_edge_scatter(g1lo, g1hi, ed4, z128, n, n_pad, e_pad)
    p1, g2lo, g2hi = _tc_layer2(s1, g1lo, g1hi, dis, b1r, W2g, batch3,
                                g_pool, bn)
    s2 = _sc_edge_scatter(g2lo, g2hi, ed4, z128, n, n_pad, e_pad)
    out = _tc_final(s2, g2lo, g2hi, dis, b2r, batch3, p1,
                    Wl1, bl1r, Wl2, bl2r, g_pool, bn)
    return out
